# initial kernel scaffold (unmeasured)
import jax
import jax.numpy as jnp
from jax import lax
from jax.experimental import pallas as pl
from jax.experimental.pallas import tpu as pltpu

N_DEV = 32
T = 512
D = 512
E_LOCAL = 2
F = 1024
T_PER = T // N_DEV
C_ROWS = T_PER + E_LOCAL


def kernel(x, router, W1, W2):
    chunk = jnp.concatenate(
        [x.astype(jnp.bfloat16), router.astype(jnp.bfloat16).T], axis=0
    )

    def body(chunk_ref, w1_ref, w2_ref, out_ref,
             gather, ybuf, comb, s1, r1, s2, r2):
        me = lax.axis_index("i")

        for off in range(1, N_DEV):
            dst = (me + off) % N_DEV
            pltpu.make_async_remote_copy(
                src_ref=chunk_ref,
                dst_ref=gather.at[me],
                send_sem=s1.at[dst],
                recv_sem=r1.at[me],
                device_id=(dst,),
                device_id_type=pl.DeviceIdType.MESH,
            ).start()
        gather[me] = chunk_ref[...]

        for off in range(1, N_DEV):
            src = (me + off) % N_DEV
            pltpu.make_async_remote_copy(
                src_ref=chunk_ref,
                dst_ref=gather.at[src],
                send_sem=s1.at[src],
                recv_sem=r1.at[src],
                device_id=(src,),
                device_id_type=pl.DeviceIdType.MESH,
            ).wait_recv()

        g = gather[...]
        xb = g[:, :T_PER, :].reshape(T, D)
        rT = g[:, T_PER:, :].reshape(N_DEV * E_LOCAL, D)

        gates = lax.dot_general(
            xb, rT, (((1,), (1,)), ((), ())),
            preferred_element_type=jnp.float32,
        )

        n_e = N_DEV * E_LOCAL
        iota = lax.broadcasted_iota(jnp.int32, (T, n_e), 1)
        m1 = jnp.max(gates, axis=1, keepdims=True)
        i1 = jnp.min(jnp.where(gates == m1, iota, n_e), axis=1)
        g2 = jnp.where(iota == i1[:, None], -1e30, gates)
        m2 = jnp.max(g2, axis=1, keepdims=True)
        i2 = jnp.min(jnp.where(g2 == m2, iota, n_e), axis=1)
        t = jnp.exp(m2 - m1)
        w_top1 = 1.0 / (1.0 + t)
        w_top2 = t / (1.0 + t)

        acc = jnp.zeros((T, D), jnp.float32)
        for l in range(E_LOCAL):
            e_id = me * E_LOCAL + l
            we = (jnp.where(i1 == e_id, w_top1[:, 0], 0.0)
                  + jnp.where(i2 == e_id, w_top2[:, 0], 0.0))
            h = jnp.dot(xb, w1_ref[l].astype(jnp.bfloat16),
                        preferred_element_type=jnp.float32)
            h = jnp.maximum(h, 0.0).astype(jnp.bfloat16)
            yl = jnp.dot(h, w2_ref[l].astype(jnp.bfloat16),
                         preferred_element_type=jnp.float32)
            acc = acc + yl * we[:, None]
        ybuf[...] = acc

        for off in range(1, N_DEV):
            dst = (me + off) % N_DEV
            pltpu.make_async_remote_copy(
                src_ref=ybuf.at[pl.ds(dst * T_PER, T_PER), :],
                dst_ref=comb.at[me],
                send_sem=s2.at[dst],
                recv_sem=r2.at[me],
                device_id=(dst,),
                device_id_type=pl.DeviceIdType.MESH,
            ).start()
        comb[me] = ybuf[pl.ds(me * T_PER, T_PER), :]

        for off in range(1, N_DEV):
            src = (me + off) % N_DEV
            pltpu.make_async_remote_copy(
                src_ref=ybuf.at[pl.ds(src * T_PER, T_PER), :],
                dst_ref=comb.at[src],
                send_sem=s2.at[src],
                recv_sem=r2.at[src],
                device_id=(src,),
                device_id_type=pl.DeviceIdType.MESH,
            ).wait_recv()

        out_ref[...] = jnp.sum(comb[...], axis=0)

        for off in range(1, N_DEV):
            dst = (me + off) % N_DEV
            pltpu.make_async_remote_copy(
                src_ref=chunk_ref,
                dst_ref=gather.at[me],
                send_sem=s1.at[dst],
                recv_sem=r1.at[me],
                device_id=(dst,),
                device_id_type=pl.DeviceIdType.MESH,
            ).wait_send()
            pltpu.make_async_remote_copy(
                src_ref=ybuf.at[pl.ds(dst * T_PER, T_PER), :],
                dst_ref=comb.at[me],
                send_sem=s2.at[dst],
                recv_sem=r2.at[me],
                device_id=(dst,),
                device_id_type=pl.DeviceIdType.MESH,
            ).wait_send()

    return pl.pallas_call(
        body,
        out_shape=jax.ShapeDtypeStruct((T_PER, D), jnp.float32),
        in_specs=[
            pl.BlockSpec(memory_space=pltpu.VMEM),
            pl.BlockSpec(memory_space=pltpu.VMEM),
            pl.BlockSpec(memory_space=pltpu.VMEM),
        ],
        out_specs=pl.BlockSpec(memory_space=pltpu.VMEM),
        scratch_shapes=[
            pltpu.VMEM((N_DEV, C_ROWS, D), jnp.bfloat16),
            pltpu.VMEM((T, D), jnp.float32),
            pltpu.VMEM((N_DEV, T_PER, D), jnp.float32),
            pltpu.SemaphoreType.DMA((N_DEV,)),
            pltpu.SemaphoreType.DMA((N_DEV,)),
            pltpu.SemaphoreType.DMA((N_DEV,)),
            pltpu.SemaphoreType.DMA((N_DEV,)),
        ],
        compiler_params=pltpu.CompilerParams(collective_id=0),
    )(chunk, W1, W2)


# baseline (device time: 54829 ns/iter reference)
import jax
import jax.numpy as jnp
from jax import lax
from jax.experimental import pallas as pl
from jax.experimental.pallas import tpu as pltpu

N_DEV = 32
T = 512
D = 512
E_LOCAL = 2
F = 1024
T_PER = T // N_DEV
C_ROWS = T_PER + E_LOCAL


def kernel(x, router, W1, W2):
    chunk = jnp.concatenate([x, router.T], axis=0)

    def body(chunk_ref, w1_ref, w2_ref, out_ref,
             gather, ybuf, comb, s1, r1, s2, r2):
        me = lax.axis_index("i")

        for off in range(1, N_DEV):
            dst = (me + off) % N_DEV
            pltpu.make_async_remote_copy(
                src_ref=chunk_ref,
                dst_ref=gather.at[me],
                send_sem=s1.at[dst],
                recv_sem=r1.at[me],
                device_id=(dst,),
                device_id_type=pl.DeviceIdType.MESH,
            ).start()
        gather[me] = chunk_ref[...]

        for off in range(1, N_DEV):
            src = (me + off) % N_DEV
            pltpu.make_async_remote_copy(
                src_ref=chunk_ref,
                dst_ref=gather.at[src],
                send_sem=s1.at[src],
                recv_sem=r1.at[src],
                device_id=(src,),
                device_id_type=pl.DeviceIdType.MESH,
            ).wait_recv()

        g = gather[...]
        xf = g[:, :T_PER, :].reshape(T, D)
        rT = g[:, T_PER:, :].reshape(N_DEV * E_LOCAL, D)

        gates = lax.dot_general(
            xf, rT, (((1,), (1,)), ((), ())),
            preferred_element_type=jnp.float32,
            precision=lax.Precision.HIGHEST,
        )
        xb = xf.astype(jnp.bfloat16)

        n_e = N_DEV * E_LOCAL
        iota = lax.broadcasted_iota(jnp.int32, (T, n_e), 1)
        m1 = jnp.max(gates, axis=1, keepdims=True)
        i1 = jnp.min(jnp.where(gates == m1, iota, n_e), axis=1)
        g2 = jnp.where(iota == i1[:, None], -1e30, gates)
        m2 = jnp.max(g2, axis=1, keepdims=True)
        i2 = jnp.min(jnp.where(g2 == m2, iota, n_e), axis=1)
        t = jnp.exp(m2 - m1)
        w_top1 = 1.0 / (1.0 + t)
        w_top2 = t / (1.0 + t)

        acc = jnp.zeros((T, D), jnp.float32)
        for l in range(E_LOCAL):
            e_id = me * E_LOCAL + l
            we = (jnp.where(i1 == e_id, w_top1[:, 0], 0.0)
                  + jnp.where(i2 == e_id, w_top2[:, 0], 0.0))
            h = jnp.dot(xb, w1_ref[l].astype(jnp.bfloat16),
                        preferred_element_type=jnp.float32)
            h = jnp.maximum(h, 0.0).astype(jnp.bfloat16)
            yl = jnp.dot(h, w2_ref[l].astype(jnp.bfloat16),
                         preferred_element_type=jnp.float32)
            acc = acc + yl * we[:, None]
        ybuf[...] = acc

        for off in range(1, N_DEV):
            dst = (me + off) % N_DEV
            pltpu.make_async_remote_copy(
                src_ref=ybuf.at[pl.ds(dst * T_PER, T_PER), :],
                dst_ref=comb.at[me],
                send_sem=s2.at[dst],
                recv_sem=r2.at[me],
                device_id=(dst,),
                device_id_type=pl.DeviceIdType.MESH,
            ).start()
        comb[me] = ybuf[pl.ds(me * T_PER, T_PER), :]

        for off in range(1, N_DEV):
            src = (me + off) % N_DEV
            pltpu.make_async_remote_copy(
                src_ref=ybuf.at[pl.ds(src * T_PER, T_PER), :],
                dst_ref=comb.at[src],
                send_sem=s2.at[src],
                recv_sem=r2.at[src],
                device_id=(src,),
                device_id_type=pl.DeviceIdType.MESH,
            ).wait_recv()

        out_ref[...] = jnp.sum(comb[...], axis=0)

        for off in range(1, N_DEV):
            dst = (me + off) % N_DEV
            pltpu.make_async_remote_copy(
                src_ref=chunk_ref,
                dst_ref=gather.at[me],
                send_sem=s1.at[dst],
                recv_sem=r1.at[me],
                device_id=(dst,),
                device_id_type=pl.DeviceIdType.MESH,
            ).wait_send()
            pltpu.make_async_remote_copy(
                src_ref=ybuf.at[pl.ds(dst * T_PER, T_PER), :],
                dst_ref=comb.at[me],
                send_sem=s2.at[dst],
                recv_sem=r2.at[me],
                device_id=(dst,),
                device_id_type=pl.DeviceIdType.MESH,
            ).wait_send()

    return pl.pallas_call(
        body,
        out_shape=jax.ShapeDtypeStruct((T_PER, D), jnp.float32),
        in_specs=[
            pl.BlockSpec(memory_space=pltpu.VMEM),
            pl.BlockSpec(memory_space=pltpu.VMEM),
            pl.BlockSpec(memory_space=pltpu.VMEM),
        ],
        out_specs=pl.BlockSpec(memory_space=pltpu.VMEM),
        scratch_shapes=[
            pltpu.VMEM((N_DEV, C_ROWS, D), jnp.float32),
            pltpu.VMEM((T, D), jnp.float32),
            pltpu.VMEM((N_DEV, T_PER, D), jnp.float32),
            pltpu.SemaphoreType.DMA((N_DEV,)),
            pltpu.SemaphoreType.DMA((N_DEV,)),
            pltpu.SemaphoreType.DMA((N_DEV,)),
            pltpu.SemaphoreType.DMA((N_DEV,)),
        ],
    )(chunk, W1, W2)


# device time: 48828 ns/iter; 1.1229x vs baseline; 1.1229x over previous
import jax
import jax.numpy as jnp
from jax import lax
from jax.experimental import pallas as pl
from jax.experimental.pallas import tpu as pltpu

N_DEV = 32
T = 512
D = 512
E_LOCAL = 2
F = 1024
T_PER = T // N_DEV
N_E = N_DEV * E_LOCAL


def kernel(x, router, W1, W2):
    xb16 = x.astype(jnp.bfloat16)
    rT = router.T

    def body(xb_ref, xf_ref, rT_ref, w1_ref, w2_ref, out_ref,
             xg, rg, sg, ssrc, ybuf, comb,
             sx, rx, sr, rr, ss, rs, s2, r2):
        me = lax.axis_index("i")

        for off in range(1, N_DEV):
            dst = (me + off) % N_DEV
            pltpu.make_async_remote_copy(
                src_ref=rT_ref, dst_ref=rg.at[me],
                send_sem=sr.at[dst], recv_sem=rr.at[me],
                device_id=(dst,), device_id_type=pl.DeviceIdType.MESH,
            ).start()
            pltpu.make_async_remote_copy(
                src_ref=xb_ref, dst_ref=xg.at[me],
                send_sem=sx.at[dst], recv_sem=rx.at[me],
                device_id=(dst,), device_id_type=pl.DeviceIdType.MESH,
            ).start()
        rg[me] = rT_ref[...]
        xg[me] = xb_ref[...]

        for off in range(1, N_DEV):
            src = (me + off) % N_DEV
            pltpu.make_async_remote_copy(
                src_ref=rT_ref, dst_ref=rg.at[src],
                send_sem=sr.at[src], recv_sem=rr.at[src],
                device_id=(src,), device_id_type=pl.DeviceIdType.MESH,
            ).wait_recv()

        rfull = rg[...].reshape(N_E, D)
        gates = lax.dot_general(
            xf_ref[...], rfull, (((1,), (1,)), ((), ())),
            preferred_element_type=jnp.float32,
            precision=lax.Precision.HIGHEST,
        )

        iota = lax.broadcasted_iota(jnp.int32, (T_PER, N_E), 1)
        m1 = jnp.max(gates, axis=1, keepdims=True)
        i1 = jnp.min(jnp.where(gates == m1, iota, N_E), axis=1, keepdims=True)
        g2 = jnp.where(iota == i1, -1e30, gates)
        m2 = jnp.max(g2, axis=1, keepdims=True)
        i2 = jnp.min(jnp.where(g2 == m2, iota, N_E), axis=1, keepdims=True)
        t = jnp.exp(m2 - m1)
        w1v = 1.0 / (1.0 + t)
        w2v = t / (1.0 + t)

        ssrc[:, :4] = jnp.concatenate(
            [i1.astype(jnp.float32), i2.astype(jnp.float32), w1v, w2v],
            axis=1,
        )

        for off in range(1, N_DEV):
            dst = (me + off) % N_DEV
            pltpu.make_async_remote_copy(
                src_ref=ssrc, dst_ref=sg.at[me],
                send_sem=ss.at[dst], recv_sem=rs.at[me],
                device_id=(dst,), device_id_type=pl.DeviceIdType.MESH,
            ).start()
        sg[me] = ssrc[...]

        for off in range(1, N_DEV):
            src = (me + off) % N_DEV
            pltpu.make_async_remote_copy(
                src_ref=xb_ref, dst_ref=xg.at[src],
                send_sem=sx.at[src], recv_sem=rx.at[src],
                device_id=(src,), device_id_type=pl.DeviceIdType.MESH,
            ).wait_recv()
            pltpu.make_async_remote_copy(
                src_ref=ssrc, dst_ref=sg.at[src],
                send_sem=ss.at[src], recv_sem=rs.at[src],
                device_id=(src,), device_id_type=pl.DeviceIdType.MESH,
            ).wait_recv()

        sa = sg[...]
        i1g = sa[:, :, 0:1].reshape(T, 1)
        i2g = sa[:, :, 1:2].reshape(T, 1)
        w1g = sa[:, :, 2:3].reshape(T, 1)
        w2g = sa[:, :, 3:4].reshape(T, 1)
        xb = xg[...].reshape(T, D)

        acc = jnp.zeros((T, D), jnp.float32)
        for l in range(E_LOCAL):
            e_f = (me * E_LOCAL + l).astype(jnp.float32)
            we = (jnp.where(i1g == e_f, w1g, 0.0)
                  + jnp.where(i2g == e_f, w2g, 0.0))
            h = jnp.dot(xb, w1_ref[l].astype(jnp.bfloat16),
                        preferred_element_type=jnp.float32)
            h = jnp.maximum(h, 0.0).astype(jnp.bfloat16)
            yl = jnp.dot(h, w2_ref[l].astype(jnp.bfloat16),
                         preferred_element_type=jnp.float32)
            acc = acc + yl * we
        ybuf[...] = acc.astype(jnp.bfloat16)

        for off in range(1, N_DEV):
            dst = (me + off) % N_DEV
            pltpu.make_async_remote_copy(
                src_ref=ybuf.at[pl.ds(dst * T_PER, T_PER), :],
                dst_ref=comb.at[me],
                send_sem=s2.at[dst], recv_sem=r2.at[me],
                device_id=(dst,), device_id_type=pl.DeviceIdType.MESH,
            ).start()
        comb[me] = ybuf[pl.ds(me * T_PER, T_PER), :]

        for off in range(1, N_DEV):
            src = (me + off) % N_DEV
            pltpu.make_async_remote_copy(
                src_ref=ybuf.at[pl.ds(src * T_PER, T_PER), :],
                dst_ref=comb.at[src],
                send_sem=s2.at[src], recv_sem=r2.at[src],
                device_id=(src,), device_id_type=pl.DeviceIdType.MESH,
            ).wait_recv()

        out_ref[...] = jnp.sum(comb[...].astype(jnp.float32), axis=0)

        for off in range(1, N_DEV):
            dst = (me + off) % N_DEV
            pltpu.make_async_remote_copy(
                src_ref=rT_ref, dst_ref=rg.at[me],
                send_sem=sr.at[dst], recv_sem=rr.at[me],
                device_id=(dst,), device_id_type=pl.DeviceIdType.MESH,
            ).wait_send()
            pltpu.make_async_remote_copy(
                src_ref=xb_ref, dst_ref=xg.at[me],
                send_sem=sx.at[dst], recv_sem=rx.at[me],
                device_id=(dst,), device_id_type=pl.DeviceIdType.MESH,
            ).wait_send()
            pltpu.make_async_remote_copy(
                src_ref=ssrc, dst_ref=sg.at[me],
                send_sem=ss.at[dst], recv_sem=rs.at[me],
                device_id=(dst,), device_id_type=pl.DeviceIdType.MESH,
            ).wait_send()
            pltpu.make_async_remote_copy(
                src_ref=ybuf.at[pl.ds(dst * T_PER, T_PER), :],
                dst_ref=comb.at[me],
                send_sem=s2.at[dst], recv_sem=r2.at[me],
                device_id=(dst,), device_id_type=pl.DeviceIdType.MESH,
            ).wait_send()

    return pl.pallas_call(
        body,
        out_shape=jax.ShapeDtypeStruct((T_PER, D), jnp.float32),
        in_specs=[pl.BlockSpec(memory_space=pltpu.VMEM)] * 5,
        out_specs=pl.BlockSpec(memory_space=pltpu.VMEM),
        scratch_shapes=[
            pltpu.VMEM((N_DEV, T_PER, D), jnp.bfloat16),
            pltpu.VMEM((N_DEV, E_LOCAL, D), jnp.float32),
            pltpu.VMEM((N_DEV, T_PER, 128), jnp.float32),
            pltpu.VMEM((T_PER, 128), jnp.float32),
            pltpu.VMEM((T, D), jnp.bfloat16),
            pltpu.VMEM((N_DEV, T_PER, D), jnp.bfloat16),
            pltpu.SemaphoreType.DMA((N_DEV,)),
            pltpu.SemaphoreType.DMA((N_DEV,)),
            pltpu.SemaphoreType.DMA((N_DEV,)),
            pltpu.SemaphoreType.DMA((N_DEV,)),
            pltpu.SemaphoreType.DMA((N_DEV,)),
            pltpu.SemaphoreType.DMA((N_DEV,)),
            pltpu.SemaphoreType.DMA((N_DEV,)),
            pltpu.SemaphoreType.DMA((N_DEV,)),
        ],
    )(xb16, x, rT, W1, W2)


# device time: 45872 ns/iter; 1.1953x vs baseline; 1.0644x over previous
import jax
import jax.numpy as jnp
from jax import lax
from jax.experimental import pallas as pl
from jax.experimental.pallas import tpu as pltpu

N_DEV = 32
T = 512
D = 512
E_LOCAL = 2
F = 1024
T_PER = T // N_DEV
N_E = N_DEV * E_LOCAL
N_CHUNK = 4
DEV_PER_CHUNK = N_DEV // N_CHUNK
T_CHUNK = T // N_CHUNK


def kernel(x, router, W1, W2):
    xb16 = x.astype(jnp.bfloat16)
    rT = router.T

    def body(xb_ref, xf_ref, rT_ref, w1_ref, w2_ref, out_ref,
             xg, rg, sg, ssrc, ybuf, comb,
             sx, rx, sr, rr, ss, rs, s2, r2):
        me = lax.axis_index("i")

        for off in range(1, N_DEV):
            dst = (me + off) % N_DEV
            pltpu.make_async_remote_copy(
                src_ref=rT_ref, dst_ref=rg.at[me],
                send_sem=sr.at[dst], recv_sem=rr.at[me],
                device_id=(dst,), device_id_type=pl.DeviceIdType.MESH,
            ).start()
            pltpu.make_async_remote_copy(
                src_ref=xb_ref, dst_ref=xg.at[me],
                send_sem=sx.at[dst], recv_sem=rx.at[me],
                device_id=(dst,), device_id_type=pl.DeviceIdType.MESH,
            ).start()
        pltpu.make_async_copy(xb_ref, xg.at[me], rx.at[me]).start()
        rg[me] = rT_ref[...]

        for off in range(1, N_DEV):
            src = (me + off) % N_DEV
            pltpu.make_async_remote_copy(
                src_ref=rT_ref, dst_ref=rg.at[src],
                send_sem=sr.at[src], recv_sem=rr.at[src],
                device_id=(src,), device_id_type=pl.DeviceIdType.MESH,
            ).wait_recv()

        rfull = rg[...].reshape(N_E, D)
        gates = lax.dot_general(
            xf_ref[...], rfull, (((1,), (1,)), ((), ())),
            preferred_element_type=jnp.float32,
            precision=lax.Precision.HIGHEST,
        )

        iota = lax.broadcasted_iota(jnp.int32, (T_PER, N_E), 1)
        m1 = jnp.max(gates, axis=1, keepdims=True)
        i1 = jnp.min(jnp.where(gates == m1, iota, N_E), axis=1, keepdims=True)
        g2 = jnp.where(iota == i1, -1e30, gates)
        m2 = jnp.max(g2, axis=1, keepdims=True)
        i2 = jnp.min(jnp.where(g2 == m2, iota, N_E), axis=1, keepdims=True)
        t = jnp.exp(m2 - m1)
        w1v = 1.0 / (1.0 + t)
        w2v = t / (1.0 + t)

        ssrc[:, :4] = jnp.concatenate(
            [i1.astype(jnp.float32), i2.astype(jnp.float32), w1v, w2v],
            axis=1,
        )

        for off in range(1, N_DEV):
            dst = (me + off) % N_DEV
            pltpu.make_async_remote_copy(
                src_ref=ssrc, dst_ref=sg.at[me],
                send_sem=ss.at[dst], recv_sem=rs.at[me],
                device_id=(dst,), device_id_type=pl.DeviceIdType.MESH,
            ).start()
        sg[me] = ssrc[...]

        for off in range(1, N_DEV):
            src = (me + off) % N_DEV
            pltpu.make_async_remote_copy(
                src_ref=ssrc, dst_ref=sg.at[src],
                send_sem=ss.at[src], recv_sem=rs.at[src],
                device_id=(src,), device_id_type=pl.DeviceIdType.MESH,
            ).wait_recv()

        sa = sg[...]
        i1g = sa[:, :, 0:1].reshape(T, 1)
        i2g = sa[:, :, 1:2].reshape(T, 1)
        w1g = sa[:, :, 2:3].reshape(T, 1)
        w2g = sa[:, :, 3:4].reshape(T, 1)

        e0 = (me * E_LOCAL).astype(jnp.float32)
        e1 = (me * E_LOCAL + 1).astype(jnp.float32)
        we0 = (jnp.where(i1g == e0, w1g, 0.0)
               + jnp.where(i2g == e0, w2g, 0.0))
        we1 = (jnp.where(i1g == e1, w1g, 0.0)
               + jnp.where(i2g == e1, w2g, 0.0))

        w1b = [w1_ref[l].astype(jnp.bfloat16) for l in range(E_LOCAL)]
        w2b = [w2_ref[l].astype(jnp.bfloat16) for l in range(E_LOCAL)]

        for c in range(N_CHUNK):
            lo_dev = c * DEV_PER_CHUNK
            for src in range(lo_dev, lo_dev + DEV_PER_CHUNK):
                pltpu.make_async_copy(xb_ref, xg.at[src], rx.at[src]).wait()

            xc = xg[lo_dev:lo_dev + DEV_PER_CHUNK].reshape(T_CHUNK, D)
            rows = slice(c * T_CHUNK, (c + 1) * T_CHUNK)
            h0 = jnp.dot(xc, w1b[0], preferred_element_type=jnp.float32)
            h0 = jnp.maximum(h0, 0.0).astype(jnp.bfloat16)
            y0 = jnp.dot(h0, w2b[0], preferred_element_type=jnp.float32)
            h1 = jnp.dot(xc, w1b[1], preferred_element_type=jnp.float32)
            h1 = jnp.maximum(h1, 0.0).astype(jnp.bfloat16)
            y1 = jnp.dot(h1, w2b[1], preferred_element_type=jnp.float32)
            acc = y0 * we0[rows] + y1 * we1[rows]
            ybuf[rows, :] = acc.astype(jnp.bfloat16)

            for d in range(lo_dev, lo_dev + DEV_PER_CHUNK):
                @pl.when(d != me)
                def _(d=d):
                    pltpu.make_async_remote_copy(
                        src_ref=ybuf.at[pl.ds(d * T_PER, T_PER), :],
                        dst_ref=comb.at[me],
                        send_sem=s2.at[d], recv_sem=r2.at[me],
                        device_id=(d,), device_id_type=pl.DeviceIdType.MESH,
                    ).start()

        comb[me] = ybuf[pl.ds(me * T_PER, T_PER), :]

        for off in range(1, N_DEV):
            src = (me + off) % N_DEV
            pltpu.make_async_remote_copy(
                src_ref=ybuf.at[pl.ds(src * T_PER, T_PER), :],
                dst_ref=comb.at[src],
                send_sem=s2.at[src], recv_sem=r2.at[src],
                device_id=(src,), device_id_type=pl.DeviceIdType.MESH,
            ).wait_recv()

        out_ref[...] = jnp.sum(comb[...].astype(jnp.float32), axis=0)

        for off in range(1, N_DEV):
            dst = (me + off) % N_DEV
            pltpu.make_async_remote_copy(
                src_ref=rT_ref, dst_ref=rg.at[me],
                send_sem=sr.at[dst], recv_sem=rr.at[me],
                device_id=(dst,), device_id_type=pl.DeviceIdType.MESH,
            ).wait_send()
            pltpu.make_async_remote_copy(
                src_ref=xb_ref, dst_ref=xg.at[me],
                send_sem=sx.at[dst], recv_sem=rx.at[me],
                device_id=(dst,), device_id_type=pl.DeviceIdType.MESH,
            ).wait_send()
            pltpu.make_async_remote_copy(
                src_ref=ssrc, dst_ref=sg.at[me],
                send_sem=ss.at[dst], recv_sem=rs.at[me],
                device_id=(dst,), device_id_type=pl.DeviceIdType.MESH,
            ).wait_send()
            pltpu.make_async_remote_copy(
                src_ref=ybuf.at[pl.ds(dst * T_PER, T_PER), :],
                dst_ref=comb.at[me],
                send_sem=s2.at[dst], recv_sem=r2.at[me],
                device_id=(dst,), device_id_type=pl.DeviceIdType.MESH,
            ).wait_send()

    return pl.pallas_call(
        body,
        out_shape=jax.ShapeDtypeStruct((T_PER, D), jnp.float32),
        in_specs=[pl.BlockSpec(memory_space=pltpu.VMEM)] * 5,
        out_specs=pl.BlockSpec(memory_space=pltpu.VMEM),
        scratch_shapes=[
            pltpu.VMEM((N_DEV, T_PER, D), jnp.bfloat16),
            pltpu.VMEM((N_DEV, E_LOCAL, D), jnp.float32),
            pltpu.VMEM((N_DEV, T_PER, 128), jnp.float32),
            pltpu.VMEM((T_PER, 128), jnp.float32),
            pltpu.VMEM((T, D), jnp.bfloat16),
            pltpu.VMEM((N_DEV, T_PER, D), jnp.bfloat16),
            pltpu.SemaphoreType.DMA((N_DEV,)),
            pltpu.SemaphoreType.DMA((N_DEV,)),
            pltpu.SemaphoreType.DMA((N_DEV,)),
            pltpu.SemaphoreType.DMA((N_DEV,)),
            pltpu.SemaphoreType.DMA((N_DEV,)),
            pltpu.SemaphoreType.DMA((N_DEV,)),
            pltpu.SemaphoreType.DMA((N_DEV,)),
            pltpu.SemaphoreType.DMA((N_DEV,)),
        ],
    )(xb16, x, rT, W1, W2)
